# Initial kernel scaffold; baseline (speedup 1.0000x reference)
import jax
import jax.numpy as jnp
from jax.experimental import pallas as pl


def kernel(x, weight, bias):
    raise NotImplementedError("write your pallas kernel here")


# speedup vs baseline: 1.5265x; 1.5265x over previous
import jax
import jax.numpy as jnp
from jax.experimental import pallas as pl
from jax.experimental.pallas import tpu as pltpu

_N = 50000
_C = 256
_EPS = 1e-05
_BR = 2000
_NB = _N // _BR
_SUB = 8


def _inorm_kernel(x_ref, w_ref, b_ref, o_ref, xs_ref, s_ref, q_ref):
    i = pl.program_id(0)

    @pl.when(i == 0)
    def _zero():
        s_ref[:] = jnp.zeros_like(s_ref)
        q_ref[:] = jnp.zeros_like(q_ref)

    @pl.when(i < _NB)
    def _accumulate():
        blk = x_ref[:]
        xs_ref[pl.ds(i * _BR, _BR), :] = blk
        g = blk.reshape(_BR // _SUB, _SUB, _C)
        s_ref[:] += jnp.sum(g, axis=0)
        q_ref[:] += jnp.sum(g * g, axis=0)

    @pl.when(i >= _NB)
    def _normalize():
        j = i - _NB
        ssum = jnp.sum(s_ref[:], axis=0, keepdims=True)
        qsum = jnp.sum(q_ref[:], axis=0, keepdims=True)
        mean = ssum * (1.0 / _N)
        var = qsum * (1.0 / _N) - mean * mean
        instd = jax.lax.rsqrt(var + _EPS)
        scale = instd * w_ref[:]
        shift = b_ref[:] - mean * scale
        o_ref[:] = xs_ref[pl.ds(j * _BR, _BR), :] * scale + shift


def kernel(x, weight, bias):
    return pl.pallas_call(
        _inorm_kernel,
        grid=(2 * _NB,),
        in_specs=[
            pl.BlockSpec((_BR, _C), lambda i: (jnp.minimum(i, _NB - 1), 0)),
            pl.BlockSpec((1, _C), lambda i: (0, 0)),
            pl.BlockSpec((1, _C), lambda i: (0, 0)),
        ],
        out_specs=pl.BlockSpec((_BR, _C), lambda i: (jnp.maximum(i - _NB, 0), 0)),
        out_shape=jax.ShapeDtypeStruct((_N, _C), jnp.float32),
        scratch_shapes=[
            pltpu.VMEM((_N, _C), jnp.float32),
            pltpu.VMEM((_SUB, _C), jnp.float32),
            pltpu.VMEM((_SUB, _C), jnp.float32),
        ],
    )(x, weight, bias)


# speedup vs baseline: 1.8997x; 1.2445x over previous
import jax
import jax.numpy as jnp
from jax.experimental import pallas as pl
from jax.experimental.pallas import tpu as pltpu

_N = 50000
_C = 256
_EPS = 1e-05
_BR = 2000
_NB = _N // _BR
_SUB = 8


def _in_copy(x_hbm, xs_ref, sems, k):
    return pltpu.make_async_copy(
        x_hbm.at[pl.ds(k * _BR, _BR), :],
        xs_ref.at[pl.ds(k * _BR, _BR), :],
        sems.at[k],
    )


def _inorm_kernel(x_hbm, w_ref, b_ref, o_ref, xs_ref, s_ref, q_ref, sems):
    i = pl.program_id(0)

    @pl.when(i == 0)
    def _start():
        s_ref[:] = jnp.zeros_like(s_ref)
        q_ref[:] = jnp.zeros_like(q_ref)
        for k in range(_NB):
            _in_copy(x_hbm, xs_ref, sems, k).start()

    @pl.when(i < _NB)
    def _accumulate():
        _in_copy(x_hbm, xs_ref, sems, i).wait()
        blk = xs_ref[pl.ds(i * _BR, _BR), :]
        g = blk.reshape(_BR // _SUB, _SUB, _C)
        s_ref[:] += jnp.sum(g, axis=0)
        q_ref[:] += jnp.sum(g * g, axis=0)

    @pl.when(i >= _NB)
    def _normalize():
        j = i - _NB
        ssum = jnp.sum(s_ref[:], axis=0, keepdims=True)
        qsum = jnp.sum(q_ref[:], axis=0, keepdims=True)
        mean = ssum * (1.0 / _N)
        var = qsum * (1.0 / _N) - mean * mean
        instd = jax.lax.rsqrt(var + _EPS)
        scale = instd * w_ref[:]
        shift = b_ref[:] - mean * scale
        o_ref[:] = xs_ref[pl.ds(j * _BR, _BR), :] * scale + shift


def kernel(x, weight, bias):
    return pl.pallas_call(
        _inorm_kernel,
        grid=(2 * _NB,),
        in_specs=[
            pl.BlockSpec(memory_space=pl.ANY),
            pl.BlockSpec((1, _C), lambda i: (0, 0)),
            pl.BlockSpec((1, _C), lambda i: (0, 0)),
        ],
        out_specs=pl.BlockSpec((_BR, _C), lambda i: (jnp.maximum(i - _NB, 0), 0)),
        out_shape=jax.ShapeDtypeStruct((_N, _C), jnp.float32),
        scratch_shapes=[
            pltpu.VMEM((_N, _C), jnp.float32),
            pltpu.VMEM((_SUB, _C), jnp.float32),
            pltpu.VMEM((_SUB, _C), jnp.float32),
            pltpu.SemaphoreType.DMA((_NB,)),
        ],
    )(x, weight, bias)


# speedup vs baseline: 2.0259x; 1.0664x over previous
import jax
import jax.numpy as jnp
from jax.experimental import pallas as pl
from jax.experimental.pallas import tpu as pltpu

_N = 50000
_C = 256
_EPS = 1e-05
_BR = 2000
_NB = _N // _BR
_SUB = 8


def _blk_copy(src, dst, sems, k):
    return pltpu.make_async_copy(
        src.at[pl.ds(k * _BR, _BR), :],
        dst.at[pl.ds(k * _BR, _BR), :],
        sems.at[k],
    )


def _inorm_kernel(x_hbm, w_ref, b_ref, o_hbm, xs_ref, s_ref, q_ref,
                  in_sems, out_sems):
    i = pl.program_id(0)

    @pl.when(i == 0)
    def _start():
        s_ref[:] = jnp.zeros_like(s_ref)
        q_ref[:] = jnp.zeros_like(q_ref)
        for k in range(_NB):
            _blk_copy(x_hbm, xs_ref, in_sems, k).start()

    @pl.when(i < _NB)
    def _accumulate():
        _blk_copy(x_hbm, xs_ref, in_sems, i).wait()
        blk = xs_ref[pl.ds(i * _BR, _BR), :]
        g = blk.reshape(_BR // _SUB, _SUB, _C)
        s_ref[:] += jnp.sum(g, axis=0)
        q_ref[:] += jnp.sum(g * g, axis=0)

    @pl.when(i >= _NB)
    def _normalize():
        j = i - _NB
        ssum = jnp.sum(s_ref[:], axis=0, keepdims=True)
        qsum = jnp.sum(q_ref[:], axis=0, keepdims=True)
        mean = ssum * (1.0 / _N)
        var = qsum * (1.0 / _N) - mean * mean
        instd = jax.lax.rsqrt(var + _EPS)
        scale = instd * w_ref[:]
        shift = b_ref[:] - mean * scale
        xs_ref[pl.ds(j * _BR, _BR), :] = (
            xs_ref[pl.ds(j * _BR, _BR), :] * scale + shift)
        _blk_copy(xs_ref, o_hbm, out_sems, j).start()

    @pl.when(i == 2 * _NB - 1)
    def _drain():
        for k in range(_NB):
            _blk_copy(xs_ref, o_hbm, out_sems, k).wait()


def kernel(x, weight, bias):
    return pl.pallas_call(
        _inorm_kernel,
        grid=(2 * _NB,),
        in_specs=[
            pl.BlockSpec(memory_space=pl.ANY),
            pl.BlockSpec((1, _C), lambda i: (0, 0)),
            pl.BlockSpec((1, _C), lambda i: (0, 0)),
        ],
        out_specs=pl.BlockSpec(memory_space=pl.ANY),
        out_shape=jax.ShapeDtypeStruct((_N, _C), jnp.float32),
        scratch_shapes=[
            pltpu.VMEM((_N, _C), jnp.float32),
            pltpu.VMEM((_SUB, _C), jnp.float32),
            pltpu.VMEM((_SUB, _C), jnp.float32),
            pltpu.SemaphoreType.DMA((_NB,)),
            pltpu.SemaphoreType.DMA((_NB,)),
        ],
    )(x, weight, bias)
